# Initial kernel scaffold; baseline (speedup 1.0000x reference)
import functools

import jax
import jax.numpy as jnp
from jax import lax
from jax.experimental import pallas as pl
from jax.experimental.pallas import tpu as pltpu
from jax.experimental.pallas import tpu_sc as plsc

N = 10000
E = 320000
D_IN = 128
HID = 64
HEADS = 4
NCLS = 40

NC = 2
NS = 16
LANES = 16

B = 80

F1 = 128
M1 = F1 + LANES
F2 = 48

ROWS_PER_TILE = N // NS


def _proj1_body(x_ref, wl_ref, wr_ref, xl_ref, xr_ref):
    xb = x_ref[...]
    l = jnp.dot(xb, wl_ref[...], preferred_element_type=jnp.float32)
    r = jnp.dot(xb, wr_ref[...], preferred_element_type=jnp.float32)
    xl_ref[0] = l[:, :F1]
    xl_ref[1] = l[:, F1:]
    xr_ref[0] = r[:, :F1]
    xr_ref[1] = r[:, F1:]


def _proj1(x, Wl1, Wr1):
    blk = 2000
    grid = N // blk
    out = jax.ShapeDtypeStruct((2, N, F1), jnp.float32)
    xl, xr = pl.pallas_call(
        _proj1_body,
        grid=(grid,),
        in_specs=[
            pl.BlockSpec((blk, D_IN), lambda i: (i, 0)),
            pl.BlockSpec((D_IN, 2 * F1), lambda i: (0, 0)),
            pl.BlockSpec((D_IN, 2 * F1), lambda i: (0, 0)),
        ],
        out_specs=[
            pl.BlockSpec((2, blk, F1), lambda i: (0, i, 0)),
            pl.BlockSpec((2, blk, F1), lambda i: (0, i, 0)),
        ],
        out_shape=[out, out],
    )(x, Wl1, Wr1)
    return xl.reshape(2 * N, F1), xr.reshape(2 * N, F1)


def _sc1_body(src_h, dst_h, xl_h, xr_h, att_h, out_h,
              src_v, dst_v, xlb, xrb, msgb, attv, accs):
    c = lax.axis_index("c")
    s = lax.axis_index("s")

    zero16 = jnp.zeros((LANES,), jnp.float32)

    @pl.loop(0, B)
    def _zero_msg(i):
        for j in range(M1 // LANES):
            msgb[i, pl.ds(LANES * j, LANES)] = zero16

    base_r = s * ROWS_PER_TILE
    for k in range(ROWS_PER_TILE // B):
        pltpu.sync_copy(msgb, accs.at[pl.ds(base_r + B * k, B)])
    rem = ROWS_PER_TILE % B
    if rem:
        pltpu.sync_copy(msgb.at[pl.ds(0, rem)],
                        accs.at[pl.ds(base_r + (ROWS_PER_TILE // B) * B, rem)])

    pltpu.sync_copy(att_h.at[c], attv)
    plsc.subcore_barrier()

    att_regs = [attv[pl.ds(LANES * j, LANES)] for j in range(F1 // LANES)]
    lane = lax.iota(jnp.int32, LANES)
    row_off = jnp.broadcast_to(c * N, (LANES,)).astype(jnp.int32)

    edges_per_tile = E // NS
    n_blocks = edges_per_tile // B

    @pl.loop(0, n_blocks)
    def _blk(it):
        base = s * edges_per_tile + it * B
        pltpu.sync_copy(src_h.at[pl.ds(base, B)], src_v)
        pltpu.sync_copy(dst_h.at[pl.ds(base, B)], dst_v)

        @pl.loop(0, B, step=LANES)
        def _off(i):
            src_v[pl.ds(i, LANES)] = src_v[pl.ds(i, LANES)] + row_off

        pltpu.sync_copy(xl_h.at[src_v], xlb)
        pltpu.sync_copy(xr_h.at[dst_v], xrb)

        @pl.loop(0, B)
        def _edge(i):
            s0 = zero16
            s1 = zero16
            xs = []
            for j in range(F1 // LANES):
                xlj = xlb[i, pl.ds(LANES * j, LANES)]
                xrj = xrb[i, pl.ds(LANES * j, LANES)]
                u = xlj + xrj
                v = jnp.where(u > 0, u, 0.2 * u)
                t = v * att_regs[j]
                if j < 4:
                    s0 = s0 + t
                else:
                    s1 = s1 + t
                xs.append(xlj)
            e0 = jnp.exp(jnp.broadcast_to(jnp.sum(s0), (LANES,)))
            e1 = jnp.exp(jnp.broadcast_to(jnp.sum(s1), (LANES,)))
            for j in range(F1 // LANES):
                msgb[i, pl.ds(LANES * j, LANES)] = (e0 if j < 4 else e1) * xs[j]
            denv = jnp.where(lane == 0, e0, jnp.where(lane == 1, e1, 0.0))
            msgb[i, pl.ds(F1, LANES)] = denv

        pltpu.sync_copy(msgb, accs.at[dst_v], add=True)

    plsc.subcore_barrier()
    pltpu.sync_copy(accs.at[pl.ds(base_r, ROWS_PER_TILE)],
                    out_h.at[pl.ds(c * N + base_r, ROWS_PER_TILE)])


def _sc1(src, dst, xl_s, xr_s, atts):
    mesh = plsc.VectorSubcoreMesh(core_axis_name="c", subcore_axis_name="s")
    f = pl.kernel(
        _sc1_body,
        out_type=jax.ShapeDtypeStruct((2 * N, M1), jnp.float32),
        mesh=mesh,
        scratch_types=[
            pltpu.VMEM((B,), jnp.int32),
            pltpu.VMEM((B,), jnp.int32),
            pltpu.VMEM((B, F1), jnp.float32),
            pltpu.VMEM((B, F1), jnp.float32),
            pltpu.VMEM((B, M1), jnp.float32),
            pltpu.VMEM((F1,), jnp.float32),
            pltpu.VMEM_SHARED((N, M1), jnp.float32),
        ],
    )
    return f(src, dst, xl_s, xr_s, atts)


def _mid_body(a0_ref, a1_ref, b1_ref, wl_ref, wr_ref, xl_ref, xr_ref):
    a0 = a0_ref[...]
    a1 = a1_ref[...]
    eps = 1e-16
    h = jnp.concatenate([
        a0[:, 0:HID] / (a0[:, F1:F1 + 1] + eps),
        a0[:, HID:F1] / (a0[:, F1 + 1:F1 + 2] + eps),
        a1[:, 0:HID] / (a1[:, F1:F1 + 1] + eps),
        a1[:, HID:F1] / (a1[:, F1 + 1:F1 + 2] + eps),
    ], axis=1) + b1_ref[...]
    h = jnp.where(h > 0, h, jnp.exp(jnp.minimum(h, 0.0)) - 1.0)
    xl_ref[...] = jnp.dot(h, wl_ref[...], preferred_element_type=jnp.float32)
    xr_ref[...] = jnp.dot(h, wr_ref[...], preferred_element_type=jnp.float32)


def _mid(acc1, b1, Wl2p, Wr2p):
    blk = 1000
    grid = N // blk
    out = jax.ShapeDtypeStruct((N, F2), jnp.float32)
    return pl.pallas_call(
        _mid_body,
        grid=(grid,),
        in_specs=[
            pl.BlockSpec((blk, M1), lambda i: (i, 0)),
            pl.BlockSpec((blk, M1), lambda i: (10 + i, 0)),
            pl.BlockSpec((1, HEADS * HID), lambda i: (0, 0)),
            pl.BlockSpec((HEADS * HID, F2), lambda i: (0, 0)),
            pl.BlockSpec((HEADS * HID, F2), lambda i: (0, 0)),
        ],
        out_specs=[
            pl.BlockSpec((blk, F2), lambda i: (i, 0)),
            pl.BlockSpec((blk, F2), lambda i: (i, 0)),
        ],
        out_shape=[out, out],
    )(acc1, acc1, b1, Wl2p, Wr2p)


def _sc2_body(src_h, dst_h, xl_h, xr_h, att_h, out_h,
              src_v, dst_v, xlb, xrb, msgb, attv, accs):
    c = lax.axis_index("c")
    s = lax.axis_index("s")

    zero16 = jnp.zeros((LANES,), jnp.float32)

    @pl.loop(0, B)
    def _zero_msg(i):
        for j in range(F2 // LANES):
            msgb[i, pl.ds(LANES * j, LANES)] = zero16

    base_r = s * ROWS_PER_TILE
    for k in range(ROWS_PER_TILE // B):
        pltpu.sync_copy(msgb, accs.at[pl.ds(base_r + B * k, B)])
    rem = ROWS_PER_TILE % B
    if rem:
        pltpu.sync_copy(msgb.at[pl.ds(0, rem)],
                        accs.at[pl.ds(base_r + (ROWS_PER_TILE // B) * B, rem)])

    pltpu.sync_copy(att_h, attv)
    plsc.subcore_barrier()

    att_regs = [attv[pl.ds(LANES * j, LANES)] for j in range(F2 // LANES)]
    lane = lax.iota(jnp.int32, LANES)
    den1 = jnp.where(lane == (NCLS % LANES), 1.0, 0.0)

    edges_per_tile = E // (NC * NS)
    n_blocks = edges_per_tile // B

    @pl.loop(0, n_blocks)
    def _blk(it):
        base = (c * NS + s) * edges_per_tile + it * B
        pltpu.sync_copy(src_h.at[pl.ds(base, B)], src_v)
        pltpu.sync_copy(dst_h.at[pl.ds(base, B)], dst_v)

        pltpu.sync_copy(xl_h.at[src_v], xlb)
        pltpu.sync_copy(xr_h.at[dst_v], xrb)

        @pl.loop(0, B)
        def _edge(i):
            acc = zero16
            xs = []
            for j in range(F2 // LANES):
                xlj = xlb[i, pl.ds(LANES * j, LANES)]
                xrj = xrb[i, pl.ds(LANES * j, LANES)]
                u = xlj + xrj
                v = jnp.where(u > 0, u, 0.2 * u)
                acc = acc + v * att_regs[j]
                xs.append(xlj)
            e = jnp.exp(jnp.broadcast_to(jnp.sum(acc), (LANES,)))
            msgb[i, pl.ds(0, LANES)] = e * xs[0]
            msgb[i, pl.ds(LANES, LANES)] = e * xs[1]
            msgb[i, pl.ds(2 * LANES, LANES)] = e * (xs[2] + den1)

        pltpu.sync_copy(msgb, accs.at[dst_v], add=True)

    plsc.subcore_barrier()
    pltpu.sync_copy(accs.at[pl.ds(base_r, ROWS_PER_TILE)],
                    out_h.at[pl.ds(c * N + base_r, ROWS_PER_TILE)])


def _sc2(src, dst, xl2p, xr2p, att2p):
    mesh = plsc.VectorSubcoreMesh(core_axis_name="c", subcore_axis_name="s")
    f = pl.kernel(
        _sc2_body,
        out_type=jax.ShapeDtypeStruct((2 * N, F2), jnp.float32),
        mesh=mesh,
        scratch_types=[
            pltpu.VMEM((B,), jnp.int32),
            pltpu.VMEM((B,), jnp.int32),
            pltpu.VMEM((B, F2), jnp.float32),
            pltpu.VMEM((B, F2), jnp.float32),
            pltpu.VMEM((B, F2), jnp.float32),
            pltpu.VMEM((F2,), jnp.float32),
            pltpu.VMEM_SHARED((N, F2), jnp.float32),
        ],
    )
    return f(src, dst, xl2p, xr2p, att2p)


def _fin_body(a0_ref, a1_ref, b2_ref, o_ref):
    num = a0_ref[...] + a1_ref[...]
    den = num[:, NCLS:NCLS + 1] + 1e-16
    o = num[:, 0:NCLS] / den + b2_ref[...]
    m = jnp.max(o, axis=1, keepdims=True)
    z = o - m
    lse = jnp.log(jnp.sum(jnp.exp(z), axis=1, keepdims=True))
    o_ref[...] = z - lse


def _fin(acc2, b2):
    blk = 1000
    grid = N // blk
    return pl.pallas_call(
        _fin_body,
        grid=(grid,),
        in_specs=[
            pl.BlockSpec((blk, F2), lambda i: (i, 0)),
            pl.BlockSpec((blk, F2), lambda i: (10 + i, 0)),
            pl.BlockSpec((1, NCLS), lambda i: (0, 0)),
        ],
        out_specs=pl.BlockSpec((blk, NCLS), lambda i: (i, 0)),
        out_shape=jax.ShapeDtypeStruct((N, NCLS), jnp.float32),
    )(acc2, acc2, b2)


def kernel(x, edge_index, Wl1, Wr1, att1, b1, Wl2, Wr2, att2, b2):
    src = edge_index[0]
    dst = edge_index[1]

    xl_s, xr_s = _proj1(x, Wl1, Wr1)
    atts = att1.reshape(2, F1)
    acc1 = _sc1(src, dst, xl_s, xr_s, atts)

    Wl2p = jnp.pad(Wl2, ((0, 0), (0, F2 - NCLS)))
    Wr2p = jnp.pad(Wr2, ((0, 0), (0, F2 - NCLS)))
    att2p = jnp.pad(att2[0], (0, F2 - NCLS))
    xl2p, xr2p = _mid(acc1, b1.reshape(1, HEADS * HID), Wl2p, Wr2p)

    acc2 = _sc2(src, dst, xl2p, xr2p, att2p)
    return _fin(acc2, b2.reshape(1, NCLS))


# speedup vs baseline: 10.1541x; 10.1541x over previous
import dataclasses

import numpy as np

import jax
import jax.numpy as jnp
from jax import lax
from jax.experimental import pallas as pl
from jax.experimental.pallas import tpu as pltpu
from jax.experimental.pallas import tpu_sc as plsc

N = 10000
E = 320000
D_IN = 128
HID = 64
HEADS = 4
NCLS = 40

NC = 2
NS = 16
LANES = 16

B = 80

F1 = 128
NH = N // 2
A1 = 5120
TR1 = 5112
DRH = 80
A2 = 2560
TR2 = 2520

ZSPAN1 = 320
ZCHUNK2 = 160


def _sc_compiler_params():
    cp = pltpu.CompilerParams()
    if "needs_layout_passes" in pltpu.CompilerParams.__dataclass_fields__:
        cp = dataclasses.replace(cp, needs_layout_passes=False)
    return cp


def _proj1_body(x_ref, wl_ref, wr_ref, xl_ref, xr_ref):
    xb = x_ref[...]
    l = jnp.dot(xb, wl_ref[...], preferred_element_type=jnp.float32)
    r = jnp.dot(xb, wr_ref[...], preferred_element_type=jnp.float32)
    xl_ref[0] = l[:, :F1]
    xl_ref[1] = l[:, F1:]
    xr_ref[0] = r[:, :F1]
    xr_ref[1] = r[:, F1:]


def _proj1(x, Wl1, Wr1):
    blk = 2000
    grid = N // blk
    out = jax.ShapeDtypeStruct((2, N, F1), jnp.float32)
    xl, xr = pl.pallas_call(
        _proj1_body,
        grid=(grid,),
        in_specs=[
            pl.BlockSpec((blk, D_IN), lambda i: (i, 0)),
            pl.BlockSpec((D_IN, 2 * F1), lambda i: (0, 0)),
            pl.BlockSpec((D_IN, 2 * F1), lambda i: (0, 0)),
        ],
        out_specs=[
            pl.BlockSpec((2, blk, F1), lambda i: (0, i, 0)),
            pl.BlockSpec((2, blk, F1), lambda i: (0, i, 0)),
        ],
        out_shape=[out, out],
    )(x, Wl1, Wr1)
    return xl.reshape(2 * N, F1), xr.reshape(2 * N, F1)


def _sc1(src, dst, xl_s, xr_s, atts, klo):
    def body(src_h, dst_h, xl_h, xr_h, att_h, out_h, den_h,
             src_v, dst_v, row_v, xlb, xrb, msgb, attv, denp, accs):
        c = lax.axis_index("c")
        s = lax.axis_index("s")

        zero16 = jnp.zeros((LANES,), jnp.float32)
        lane = lax.iota(jnp.int32, LANES)

        @pl.loop(0, B)
        def _zero_msg(i):
            for j in range(F1 // LANES):
                msgb[i, pl.ds(LANES * j, LANES)] = zero16

        @pl.loop(0, DRH)
        def _zero_den(i):
            for j in range(F1 // LANES):
                denp[i, pl.ds(LANES * j, LANES)] = zero16

        base_r = s * ZSPAN1
        for k in range(ZSPAN1 // B):
            pltpu.sync_copy(msgb, accs.at[pl.ds(base_r + B * k, B)])

        pltpu.sync_copy(att_h.at[pl.ds(c * F1, F1)], attv)
        plsc.subcore_barrier()

        att_regs = [attv[pl.ds(LANES * j, LANES)] for j in range(F1 // LANES)]
        tab_off = jnp.broadcast_to(c * N, (LANES,)).astype(jnp.int32)
        node_lo = jnp.broadcast_to(klo, (LANES,)).astype(jnp.int32)
        den_mask = lane < 2
        trash16 = jnp.broadcast_to(TR1, (LANES,)).astype(jnp.int32)

        edges_per_tile = E // NS
        n_blocks = edges_per_tile // B

        @pl.loop(0, n_blocks)
        def _blk(it):
            base = s * edges_per_tile + it * B
            pltpu.sync_copy(src_h.at[pl.ds(base, B)], src_v)
            pltpu.sync_copy(dst_h.at[pl.ds(base, B)], dst_v)

            @pl.loop(0, B, step=LANES)
            def _route(i):
                src_v[pl.ds(i, LANES)] = src_v[pl.ds(i, LANES)] + tab_off
                local = dst_v[pl.ds(i, LANES)] - node_lo
                inr = jnp.logical_and(local >= 0, local < NH)
                row_v[pl.ds(i, LANES)] = jnp.where(inr, local, trash16)

            pltpu.sync_copy(xl_h.at[src_v], xlb)
            pltpu.sync_copy(xr_h.at[dst_v], xrb)

            @pl.loop(0, B)
            def _edge(i):
                s0 = zero16
                s1 = zero16
                xs = []
                for j in range(F1 // LANES):
                    xlj = xlb[i, pl.ds(LANES * j, LANES)]
                    xrj = xrb[i, pl.ds(LANES * j, LANES)]
                    u = xlj + xrj
                    v = jnp.where(u > 0, u, 0.2 * u)
                    t = v * att_regs[j]
                    if j < 4:
                        s0 = s0 + t
                    else:
                        s1 = s1 + t
                    xs.append(xlj)
                e0 = jnp.exp(jnp.broadcast_to(jnp.sum(s0), (LANES,)))
                e1 = jnp.exp(jnp.broadcast_to(jnp.sum(s1), (LANES,)))
                for j in range(F1 // LANES):
                    msgb[i, pl.ds(LANES * j, LANES)] = (
                        (e0 if j < 4 else e1) * xs[j])
                lcl = plsc.load_gather(
                    row_v, [jnp.broadcast_to(i, (LANES,))])
                ok = lcl < NH
                fl = 2 * jnp.where(ok, lcl, 0) + lane
                denv = jnp.where(lane == 0, e0, e1)
                plsc.addupdate_scatter(
                    denp, [lax.shift_right_logical(fl, 7),
                           jnp.bitwise_and(fl, 127)], denv,
                    mask=jnp.logical_and(den_mask, ok))

            pltpu.sync_copy(msgb, accs.at[row_v], add=True)

        plsc.subcore_barrier()

        pltpu.sync_copy(accs.at[pl.ds(base_r, ZSPAN1)],
                        out_h.at[pl.ds(c * A1 + base_r, ZSPAN1)])
        pltpu.sync_copy(denp, den_h.at[pl.ds((c * NS + s) * DRH, DRH)])

    mesh = plsc.VectorSubcoreMesh(core_axis_name="c", subcore_axis_name="s")
    f = pl.kernel(
        body,
        out_type=[jax.ShapeDtypeStruct((2 * A1, F1), jnp.float32),
                  jax.ShapeDtypeStruct((2 * NS * DRH, F1), jnp.float32)],
        mesh=mesh,
        compiler_params=_sc_compiler_params(),
        scratch_types=[
            pltpu.VMEM((B,), jnp.int32),
            pltpu.VMEM((B,), jnp.int32),
            pltpu.VMEM((B,), jnp.int32),
            pltpu.VMEM((B, F1), jnp.float32),
            pltpu.VMEM((B, F1), jnp.float32),
            pltpu.VMEM((B, F1), jnp.float32),
            pltpu.VMEM((F1,), jnp.float32),
            pltpu.VMEM((DRH, F1), jnp.float32),
            pltpu.VMEM_SHARED((A1, F1), jnp.float32),
        ],
    )
    return f(src, dst, xl_s, xr_s, atts)


def _dmerge_body(*refs):
    ins, out = refs[:-1], refs[-1]
    acc = ins[0][...]
    for r in ins[1:]:
        acc = acc + r[...]
    out[...] = acc


def _dmerge(den_all):
    specs = [
        pl.BlockSpec((DRH, F1), (lambda c, j=j: (c * NS + j, 0)))
        for j in range(NS)
    ]
    return pl.pallas_call(
        _dmerge_body,
        grid=(2,),
        in_specs=specs,
        out_specs=pl.BlockSpec((DRH, F1), lambda c: (c, 0)),
        out_shape=jax.ShapeDtypeStruct((2 * DRH, F1), jnp.float32),
    )(*([den_all] * NS))


def _mid_body(a0_ref, a1_ref, d0_ref, d1_ref, b1_ref, wl_ref, wr_ref,
              xl_ref, xr_ref):
    a0 = a0_ref[...]
    a1 = a1_ref[...]
    d0 = d0_ref[...]
    d1 = d1_ref[...]
    eps = 1e-16
    h = jnp.concatenate([
        a0[:, 0:HID] / (d0[:, 0:1] + eps),
        a0[:, HID:F1] / (d0[:, 1:2] + eps),
        a1[:, 0:HID] / (d1[:, 0:1] + eps),
        a1[:, HID:F1] / (d1[:, 1:2] + eps),
    ], axis=1) + b1_ref[...]
    h = jnp.where(h > 0, h, jnp.exp(jnp.minimum(h, 0.0)) - 1.0)
    xl_ref[...] = jnp.dot(h, wl_ref[...], preferred_element_type=jnp.float32)
    xr_ref[...] = jnp.dot(h, wr_ref[...], preferred_element_type=jnp.float32)


def _mid(a0, a1, d0, d1, b1, Wl2p, Wr2p):
    blk = 1000
    grid = N // blk
    out = jax.ShapeDtypeStruct((N, F1), jnp.float32)
    wspec = pl.BlockSpec((HEADS * HID, F1), lambda i: (0, 0))
    return pl.pallas_call(
        _mid_body,
        grid=(grid,),
        in_specs=[
            pl.BlockSpec((blk, F1), lambda i: (i, 0)),
            pl.BlockSpec((blk, F1), lambda i: (i, 0)),
            pl.BlockSpec((blk, 2), lambda i: (i, 0)),
            pl.BlockSpec((blk, 2), lambda i: (i, 0)),
            pl.BlockSpec((1, HEADS * HID), lambda i: (0, 0)),
            wspec, wspec,
        ],
        out_specs=[
            pl.BlockSpec((blk, F1), lambda i: (i, 0)),
            pl.BlockSpec((blk, F1), lambda i: (i, 0)),
        ],
        out_shape=[out, out],
    )(a0, a1, d0, d1, b1, Wl2p, Wr2p)


def _sc2_body(src_h, dst_h, xl_h, xr_h, att_h, out_h,
              src_v, dst_v, row_v, xlb, xrb, msgb, attv, accs):
    c = lax.axis_index("c")
    s = lax.axis_index("s")

    zero16 = jnp.zeros((LANES,), jnp.float32)
    lane = lax.iota(jnp.int32, LANES)

    @pl.loop(0, B)
    def _zero_msg(i):
        for j in range(F1 // LANES):
            msgb[i, pl.ds(LANES * j, LANES)] = zero16

    base_r = s * ZCHUNK2
    for k in range(ZCHUNK2 // B):
        pltpu.sync_copy(msgb, accs.at[pl.ds(base_r + B * k, B)])

    pltpu.sync_copy(att_h, attv)
    plsc.subcore_barrier()

    natt = 48 // LANES
    att_regs = [attv[pl.ds(LANES * j, LANES)] for j in range(natt)]
    den1 = jnp.where(lane == (NCLS % LANES), 1.0, 0.0)
    node_lo = jnp.broadcast_to(c * NH, (LANES,)).astype(jnp.int32)
    trash16 = jnp.broadcast_to(TR2, (LANES,)).astype(jnp.int32)

    edges_per_tile = E // NS
    n_blocks = edges_per_tile // B

    @pl.loop(0, n_blocks)
    def _blk(it):
        base = s * edges_per_tile + it * B
        pltpu.sync_copy(src_h.at[pl.ds(base, B)], src_v)
        pltpu.sync_copy(dst_h.at[pl.ds(base, B)], dst_v)

        @pl.loop(0, B, step=LANES)
        def _route(i):
            local = dst_v[pl.ds(i, LANES)] - node_lo
            inr = jnp.logical_and(local >= 0, local < NH)
            row_v[pl.ds(i, LANES)] = jnp.where(
                inr, lax.shift_right_logical(local, 1), trash16)

        pltpu.sync_copy(xl_h.at[src_v], xlb)
        pltpu.sync_copy(xr_h.at[dst_v], xrb)

        @pl.loop(0, B)
        def _edge(i):
            acc = zero16
            xs = []
            for j in range(natt):
                xlj = xlb[i, pl.ds(LANES * j, LANES)]
                xrj = xrb[i, pl.ds(LANES * j, LANES)]
                u = xlj + xrj
                v = jnp.where(u > 0, u, 0.2 * u)
                acc = acc + v * att_regs[j]
                xs.append(xlj)
            e = jnp.exp(jnp.broadcast_to(jnp.sum(acc), (LANES,)))
            m0 = e * xs[0]
            m1 = e * xs[1]
            m2 = e * (xs[2] + den1)
            d = plsc.load_gather(dst_v, [jnp.broadcast_to(i, (LANES,))])
            odd = jnp.bitwise_and(d, 1) == 1
            for j, m in enumerate((m0, m1, m2)):
                msgb[i, pl.ds(LANES * j, LANES)] = jnp.where(odd, 0.0, m)
                msgb[i, pl.ds(64 + LANES * j, LANES)] = jnp.where(odd, m, 0.0)

        pltpu.sync_copy(msgb, accs.at[row_v], add=True)

    plsc.subcore_barrier()
    pltpu.sync_copy(accs.at[pl.ds(base_r, ZCHUNK2)],
                    out_h.at[pl.ds(c * A2 + base_r, ZCHUNK2)])


def _sc2(src, dst, xl2p, xr2p, att2p):
    mesh = plsc.VectorSubcoreMesh(core_axis_name="c", subcore_axis_name="s")
    f = pl.kernel(
        _sc2_body,
        out_type=jax.ShapeDtypeStruct((2 * A2, F1), jnp.float32),
        mesh=mesh,
        compiler_params=_sc_compiler_params(),
        scratch_types=[
            pltpu.VMEM((B,), jnp.int32),
            pltpu.VMEM((B,), jnp.int32),
            pltpu.VMEM((B,), jnp.int32),
            pltpu.VMEM((B, F1), jnp.float32),
            pltpu.VMEM((B, F1), jnp.float32),
            pltpu.VMEM((B, F1), jnp.float32),
            pltpu.VMEM((F1,), jnp.float32),
            pltpu.VMEM_SHARED((A2, F1), jnp.float32),
        ],
    )
    return f(src, dst, xl2p, xr2p, att2p)


def _fin_body(a_ref, b2_ref, o_ref):
    num = a_ref[...]
    den = num[:, NCLS:NCLS + 1] + 1e-16
    o = num[:, 0:NCLS] / den + b2_ref[...]
    m = jnp.max(o, axis=1, keepdims=True)
    z = o - m
    lse = jnp.log(jnp.sum(jnp.exp(z), axis=1, keepdims=True))
    o_ref[...] = z - lse


def _fin(acc2, b2):
    blk = 1000
    grid = N // blk
    return pl.pallas_call(
        _fin_body,
        grid=(grid,),
        in_specs=[
            pl.BlockSpec((blk, 64), lambda i: (i, 0)),
            pl.BlockSpec((1, NCLS), lambda i: (0, 0)),
        ],
        out_specs=pl.BlockSpec((blk, NCLS), lambda i: (i, 0)),
        out_shape=jax.ShapeDtypeStruct((N, NCLS), jnp.float32),
    )(acc2, b2)


def kernel(x, edge_index, Wl1, Wr1, att1, b1, Wl2, Wr2, att2, b2):
    src = edge_index[0]
    dst = edge_index[1]

    xl_s, xr_s = _proj1(x, Wl1, Wr1)
    atts = att1.reshape(2 * F1)

    msgA, denA = _sc1(src, dst, xl_s, xr_s, atts, 0)
    msgB, denB = _sc1(src, dst, xl_s, xr_s, atts, NH)
    dmA = _dmerge(denA)
    dmB = _dmerge(denB)

    a0 = jnp.concatenate([msgA[:NH], msgB[:NH]], axis=0)
    a1 = jnp.concatenate([msgA[A1:A1 + NH], msgB[A1:A1 + NH]], axis=0)
    d0 = jnp.concatenate([
        dmA[:DRH].reshape(-1)[:2 * NH].reshape(NH, 2),
        dmB[:DRH].reshape(-1)[:2 * NH].reshape(NH, 2)], axis=0)
    d1 = jnp.concatenate([
        dmA[DRH:].reshape(-1)[:2 * NH].reshape(NH, 2),
        dmB[DRH:].reshape(-1)[:2 * NH].reshape(NH, 2)], axis=0)

    Wl2p = jnp.pad(Wl2, ((0, 0), (0, F1 - NCLS)))
    Wr2p = jnp.pad(Wr2, ((0, 0), (0, F1 - NCLS)))
    att2p = jnp.pad(att2[0], (0, F1 - NCLS))
    xl2p, xr2p = _mid(a0, a1, d0, d1,
                      b1.reshape(1, HEADS * HID), Wl2p, Wr2p)

    acc2 = _sc2(src, dst, xl2p, xr2p, att2p)
    o = jnp.concatenate([
        acc2[0:NH // 2].reshape(NH, 64),
        acc2[A2:A2 + NH // 2].reshape(NH, 64),
    ], axis=0)
    return _fin(o, b2.reshape(1, NCLS))


# speedup vs baseline: 14.5610x; 1.4340x over previous
import dataclasses

import numpy as np

import jax
import jax.numpy as jnp
from jax import lax
from jax.experimental import pallas as pl
from jax.experimental.pallas import tpu as pltpu
from jax.experimental.pallas import tpu_sc as plsc

N = 10000
E = 320000
D_IN = 128
HID = 64
HEADS = 4
NCLS = 40

NC = 2
NS = 16
LANES = 16

B = 80

F1 = 128
NH = N // 2
A1 = 5120
TR1 = 5112
DRH = 80
A2 = 2560
TR2 = 2520

ZSPAN1 = 320
ZCHUNK2 = 160


def _sc_compiler_params():
    cp = pltpu.CompilerParams()
    if "needs_layout_passes" in pltpu.CompilerParams.__dataclass_fields__:
        cp = dataclasses.replace(cp, needs_layout_passes=False)
    return cp


def _proj1_body(x_ref, wl_ref, wr_ref, xl_ref, xr_ref):
    xb = x_ref[...]
    l = jnp.dot(xb, wl_ref[...], preferred_element_type=jnp.float32)
    r = jnp.dot(xb, wr_ref[...], preferred_element_type=jnp.float32)
    xl_ref[0] = l[:, :F1]
    xl_ref[1] = l[:, F1:]
    xr_ref[0] = r[:, :F1]
    xr_ref[1] = r[:, F1:]


def _proj1(x, Wl1, Wr1):
    blk = 2000
    grid = N // blk
    out = jax.ShapeDtypeStruct((2, N, F1), jnp.float32)
    xl, xr = pl.pallas_call(
        _proj1_body,
        grid=(grid,),
        in_specs=[
            pl.BlockSpec((blk, D_IN), lambda i: (i, 0)),
            pl.BlockSpec((D_IN, 2 * F1), lambda i: (0, 0)),
            pl.BlockSpec((D_IN, 2 * F1), lambda i: (0, 0)),
        ],
        out_specs=[
            pl.BlockSpec((2, blk, F1), lambda i: (0, i, 0)),
            pl.BlockSpec((2, blk, F1), lambda i: (0, i, 0)),
        ],
        out_shape=[out, out],
    )(x, Wl1, Wr1)
    return xl.reshape(2 * N, F1), xr.reshape(2 * N, F1)


def _sc1(src, dst, xl_s, xr_s, atts, klo):
    def body(src_h, dst_h, xl_h, xr_h, att_h, out_h, den_h,
             src_v, dst_v, row_v, xlb, xrb, msgb, attv, denp, accs):
        c = lax.axis_index("c")
        s = lax.axis_index("s")

        zero16 = jnp.zeros((LANES,), jnp.float32)
        lane = lax.iota(jnp.int32, LANES)

        @pl.loop(0, B)
        def _zero_msg(i):
            for j in range(F1 // LANES):
                msgb[i, pl.ds(LANES * j, LANES)] = zero16

        @pl.loop(0, DRH)
        def _zero_den(i):
            for j in range(F1 // LANES):
                denp[i, pl.ds(LANES * j, LANES)] = zero16

        base_r = s * ZSPAN1
        for k in range(ZSPAN1 // B):
            pltpu.sync_copy(msgb, accs.at[pl.ds(base_r + B * k, B)])

        pltpu.sync_copy(att_h.at[pl.ds(c * F1, F1)], attv)
        plsc.subcore_barrier()

        att_regs = [attv[pl.ds(LANES * j, LANES)] for j in range(F1 // LANES)]
        tab_off = jnp.broadcast_to(c * N, (LANES,)).astype(jnp.int32)
        node_lo = jnp.broadcast_to(klo, (LANES,)).astype(jnp.int32)
        den_mask = lane < 2
        trash16 = jnp.broadcast_to(TR1, (LANES,)).astype(jnp.int32)

        edges_per_tile = E // NS
        n_blocks = edges_per_tile // B

        @pl.loop(0, n_blocks)
        def _blk(it):
            base = s * edges_per_tile + it * B
            pltpu.sync_copy(src_h.at[pl.ds(base, B)], src_v)
            pltpu.sync_copy(dst_h.at[pl.ds(base, B)], dst_v)

            @pl.loop(0, B, step=LANES)
            def _route(i):
                src_v[pl.ds(i, LANES)] = src_v[pl.ds(i, LANES)] + tab_off
                local = dst_v[pl.ds(i, LANES)] - node_lo
                inr = jnp.logical_and(local >= 0, local < NH)
                row_v[pl.ds(i, LANES)] = jnp.where(inr, local, trash16)

            pltpu.sync_copy(xl_h.at[src_v], xlb)
            pltpu.sync_copy(xr_h.at[dst_v], xrb)

            @plsc.parallel_loop(0, B, 1, unroll=2)
            def _edge(i):
                s0 = zero16
                s1 = zero16
                xs = []
                for j in range(F1 // LANES):
                    xlj = xlb[i, pl.ds(LANES * j, LANES)]
                    xrj = xrb[i, pl.ds(LANES * j, LANES)]
                    u = xlj + xrj
                    v = jnp.where(u > 0, u, 0.2 * u)
                    t = v * att_regs[j]
                    if j < 4:
                        s0 = s0 + t
                    else:
                        s1 = s1 + t
                    xs.append(xlj)
                e0 = jnp.exp(jnp.broadcast_to(jnp.sum(s0), (LANES,)))
                e1 = jnp.exp(jnp.broadcast_to(jnp.sum(s1), (LANES,)))
                for j in range(F1 // LANES):
                    msgb[i, pl.ds(LANES * j, LANES)] = (
                        (e0 if j < 4 else e1) * xs[j])
                lcl = plsc.load_gather(
                    row_v, [jnp.broadcast_to(i, (LANES,))])
                ok = lcl < NH
                fl = 2 * jnp.where(ok, lcl, 0) + lane
                denv = jnp.where(lane == 0, e0, e1)
                plsc.addupdate_scatter(
                    denp, [lax.shift_right_logical(fl, 7),
                           jnp.bitwise_and(fl, 127)], denv,
                    mask=jnp.logical_and(den_mask, ok))

            pltpu.sync_copy(msgb, accs.at[row_v], add=True)

        plsc.subcore_barrier()

        pltpu.sync_copy(accs.at[pl.ds(base_r, ZSPAN1)],
                        out_h.at[pl.ds(c * A1 + base_r, ZSPAN1)])
        pltpu.sync_copy(denp, den_h.at[pl.ds((c * NS + s) * DRH, DRH)])

    mesh = plsc.VectorSubcoreMesh(core_axis_name="c", subcore_axis_name="s")
    f = pl.kernel(
        body,
        out_type=[jax.ShapeDtypeStruct((2 * A1, F1), jnp.float32),
                  jax.ShapeDtypeStruct((2 * NS * DRH, F1), jnp.float32)],
        mesh=mesh,
        compiler_params=_sc_compiler_params(),
        scratch_types=[
            pltpu.VMEM((B,), jnp.int32),
            pltpu.VMEM((B,), jnp.int32),
            pltpu.VMEM((B,), jnp.int32),
            pltpu.VMEM((B, F1), jnp.float32),
            pltpu.VMEM((B, F1), jnp.float32),
            pltpu.VMEM((B, F1), jnp.float32),
            pltpu.VMEM((F1,), jnp.float32),
            pltpu.VMEM((DRH, F1), jnp.float32),
            pltpu.VMEM_SHARED((A1, F1), jnp.float32),
        ],
    )
    return f(src, dst, xl_s, xr_s, atts)


def _dmerge_body(*refs):
    ins, out = refs[:-1], refs[-1]
    acc = ins[0][...]
    for r in ins[1:]:
        acc = acc + r[...]
    out[...] = acc


def _dmerge(den_all):
    specs = [
        pl.BlockSpec((DRH, F1), (lambda c, j=j: (c * NS + j, 0)))
        for j in range(NS)
    ]
    return pl.pallas_call(
        _dmerge_body,
        grid=(2,),
        in_specs=specs,
        out_specs=pl.BlockSpec((DRH, F1), lambda c: (c, 0)),
        out_shape=jax.ShapeDtypeStruct((2 * DRH, F1), jnp.float32),
    )(*([den_all] * NS))


def _mid_body(a0_ref, a1_ref, d0_ref, d1_ref, b1_ref, wl_ref, wr_ref,
              xl_ref, xr_ref):
    a0 = a0_ref[...]
    a1 = a1_ref[...]
    d0 = d0_ref[...]
    d1 = d1_ref[...]
    eps = 1e-16
    h = jnp.concatenate([
        a0[:, 0:HID] / (d0[:, 0:1] + eps),
        a0[:, HID:F1] / (d0[:, 1:2] + eps),
        a1[:, 0:HID] / (d1[:, 0:1] + eps),
        a1[:, HID:F1] / (d1[:, 1:2] + eps),
    ], axis=1) + b1_ref[...]
    h = jnp.where(h > 0, h, jnp.exp(jnp.minimum(h, 0.0)) - 1.0)
    xl_ref[...] = jnp.dot(h, wl_ref[...], preferred_element_type=jnp.float32)
    xr_ref[...] = jnp.dot(h, wr_ref[...], preferred_element_type=jnp.float32)


def _mid(a0, a1, d0, d1, b1, Wl2p, Wr2p):
    blk = 1000
    grid = N // blk
    out = jax.ShapeDtypeStruct((N, F1), jnp.float32)
    wspec = pl.BlockSpec((HEADS * HID, F1), lambda i: (0, 0))
    return pl.pallas_call(
        _mid_body,
        grid=(grid,),
        in_specs=[
            pl.BlockSpec((blk, F1), lambda i: (i, 0)),
            pl.BlockSpec((blk, F1), lambda i: (i, 0)),
            pl.BlockSpec((blk, 2), lambda i: (i, 0)),
            pl.BlockSpec((blk, 2), lambda i: (i, 0)),
            pl.BlockSpec((1, HEADS * HID), lambda i: (0, 0)),
            wspec, wspec,
        ],
        out_specs=[
            pl.BlockSpec((blk, F1), lambda i: (i, 0)),
            pl.BlockSpec((blk, F1), lambda i: (i, 0)),
        ],
        out_shape=[out, out],
    )(a0, a1, d0, d1, b1, Wl2p, Wr2p)


def _sc2_body(src_h, dst_h, xl_h, xr_h, att_h, out_h,
              src_v, dst_v, row_v, xlb, xrb, msgb, attv, accs):
    c = lax.axis_index("c")
    s = lax.axis_index("s")

    zero16 = jnp.zeros((LANES,), jnp.float32)
    lane = lax.iota(jnp.int32, LANES)

    @pl.loop(0, B)
    def _zero_msg(i):
        for j in range(F1 // LANES):
            msgb[i, pl.ds(LANES * j, LANES)] = zero16

    base_r = s * ZCHUNK2
    for k in range(ZCHUNK2 // B):
        pltpu.sync_copy(msgb, accs.at[pl.ds(base_r + B * k, B)])

    pltpu.sync_copy(att_h, attv)
    plsc.subcore_barrier()

    natt = 48 // LANES
    att_regs = [attv[pl.ds(LANES * j, LANES)] for j in range(natt)]
    den1 = jnp.where(lane == (NCLS % LANES), 1.0, 0.0)
    node_lo = jnp.broadcast_to(c * NH, (LANES,)).astype(jnp.int32)
    trash16 = jnp.broadcast_to(TR2, (LANES,)).astype(jnp.int32)

    edges_per_tile = E // NS
    n_blocks = edges_per_tile // B

    @pl.loop(0, n_blocks)
    def _blk(it):
        base = s * edges_per_tile + it * B
        pltpu.sync_copy(src_h.at[pl.ds(base, B)], src_v)
        pltpu.sync_copy(dst_h.at[pl.ds(base, B)], dst_v)

        @pl.loop(0, B, step=LANES)
        def _route(i):
            local = dst_v[pl.ds(i, LANES)] - node_lo
            inr = jnp.logical_and(local >= 0, local < NH)
            row_v[pl.ds(i, LANES)] = jnp.where(
                inr, lax.shift_right_logical(local, 1), trash16)

        pltpu.sync_copy(xl_h.at[src_v], xlb)
        pltpu.sync_copy(xr_h.at[dst_v], xrb)

        @plsc.parallel_loop(0, B, 1, unroll=2)
        def _edge(i):
            acc = zero16
            xs = []
            for j in range(natt):
                xlj = xlb[i, pl.ds(LANES * j, LANES)]
                xrj = xrb[i, pl.ds(LANES * j, LANES)]
                u = xlj + xrj
                v = jnp.where(u > 0, u, 0.2 * u)
                acc = acc + v * att_regs[j]
                xs.append(xlj)
            e = jnp.exp(jnp.broadcast_to(jnp.sum(acc), (LANES,)))
            m0 = e * xs[0]
            m1 = e * xs[1]
            m2 = e * (xs[2] + den1)
            d = plsc.load_gather(dst_v, [jnp.broadcast_to(i, (LANES,))])
            odd = jnp.bitwise_and(d, 1) == 1
            for j, m in enumerate((m0, m1, m2)):
                msgb[i, pl.ds(LANES * j, LANES)] = jnp.where(odd, 0.0, m)
                msgb[i, pl.ds(64 + LANES * j, LANES)] = jnp.where(odd, m, 0.0)

        pltpu.sync_copy(msgb, accs.at[row_v], add=True)

    plsc.subcore_barrier()
    pltpu.sync_copy(accs.at[pl.ds(base_r, ZCHUNK2)],
                    out_h.at[pl.ds(c * A2 + base_r, ZCHUNK2)])


def _sc2(src, dst, xl2p, xr2p, att2p):
    mesh = plsc.VectorSubcoreMesh(core_axis_name="c", subcore_axis_name="s")
    f = pl.kernel(
        _sc2_body,
        out_type=jax.ShapeDtypeStruct((2 * A2, F1), jnp.float32),
        mesh=mesh,
        compiler_params=_sc_compiler_params(),
        scratch_types=[
            pltpu.VMEM((B,), jnp.int32),
            pltpu.VMEM((B,), jnp.int32),
            pltpu.VMEM((B,), jnp.int32),
            pltpu.VMEM((B, F1), jnp.float32),
            pltpu.VMEM((B, F1), jnp.float32),
            pltpu.VMEM((B, F1), jnp.float32),
            pltpu.VMEM((F1,), jnp.float32),
            pltpu.VMEM_SHARED((A2, F1), jnp.float32),
        ],
    )
    return f(src, dst, xl2p, xr2p, att2p)


def _fin_body(a_ref, b2_ref, o_ref):
    num = a_ref[...]
    den = num[:, NCLS:NCLS + 1] + 1e-16
    o = num[:, 0:NCLS] / den + b2_ref[...]
    m = jnp.max(o, axis=1, keepdims=True)
    z = o - m
    lse = jnp.log(jnp.sum(jnp.exp(z), axis=1, keepdims=True))
    o_ref[...] = z - lse


def _fin(acc2, b2):
    blk = 1000
    grid = N // blk
    return pl.pallas_call(
        _fin_body,
        grid=(grid,),
        in_specs=[
            pl.BlockSpec((blk, 64), lambda i: (i, 0)),
            pl.BlockSpec((1, NCLS), lambda i: (0, 0)),
        ],
        out_specs=pl.BlockSpec((blk, NCLS), lambda i: (i, 0)),
        out_shape=jax.ShapeDtypeStruct((N, NCLS), jnp.float32),
    )(acc2, b2)


def kernel(x, edge_index, Wl1, Wr1, att1, b1, Wl2, Wr2, att2, b2):
    src = edge_index[0]
    dst = edge_index[1]

    xl_s, xr_s = _proj1(x, Wl1, Wr1)
    atts = att1.reshape(2 * F1)

    msgA, denA = _sc1(src, dst, xl_s, xr_s, atts, 0)
    msgB, denB = _sc1(src, dst, xl_s, xr_s, atts, NH)
    dmA = _dmerge(denA)
    dmB = _dmerge(denB)

    a0 = jnp.concatenate([msgA[:NH], msgB[:NH]], axis=0)
    a1 = jnp.concatenate([msgA[A1:A1 + NH], msgB[A1:A1 + NH]], axis=0)
    d0 = jnp.concatenate([
        dmA[:DRH].reshape(-1)[:2 * NH].reshape(NH, 2),
        dmB[:DRH].reshape(-1)[:2 * NH].reshape(NH, 2)], axis=0)
    d1 = jnp.concatenate([
        dmA[DRH:].reshape(-1)[:2 * NH].reshape(NH, 2),
        dmB[DRH:].reshape(-1)[:2 * NH].reshape(NH, 2)], axis=0)

    Wl2p = jnp.pad(Wl2, ((0, 0), (0, F1 - NCLS)))
    Wr2p = jnp.pad(Wr2, ((0, 0), (0, F1 - NCLS)))
    att2p = jnp.pad(att2[0], (0, F1 - NCLS))
    xl2p, xr2p = _mid(a0, a1, d0, d1,
                      b1.reshape(1, HEADS * HID), Wl2p, Wr2p)

    acc2 = _sc2(src, dst, xl2p, xr2p, att2p)
    o = jnp.concatenate([
        acc2[0:NH // 2].reshape(NH, 64),
        acc2[A2:A2 + NH // 2].reshape(NH, 64),
    ], axis=0)
    return _fin(o, b2.reshape(1, NCLS))
